# baseline (speedup 1.0000x reference)
import functools

import jax
import jax.numpy as jnp
from jax import lax
from jax.experimental import pallas as pl
from jax.experimental.pallas import tpu as pltpu
from jax.experimental.pallas import tpu_sc as plsc

NME = 65536
NNZ_PER = 64
NNZ = NME * NNZ_PER
NDOF = 132098
NDOF_PAD = 132608
EMIN = 1e-09
EMAX = 1.0
PENAL = 3.0
VOLFRAC = 0.4

NC = 2
NS = 16
NW = NC * NS
L = 16

ROWS_TOTAL = NNZ // 128
ROWS_PER_W = ROWS_TOTAL // NW
CH_ROWS = 64
N_CH = ROWS_PER_W // CH_ROWS
E_PER_W = NME // NW
DOF_SLICE = NDOF_PAD // NS


def _sc_kernel(wx_hbm, ksep_hbm, ik_hbm, u_hbm,
               ku_out, rho_out,
               wx_v, scale_v,
               ksep0, ic0, rows0, cols0, uv0,
               ksep1, ic1, rows1, cols1, uv1,
               zbuf_v, rho_v, ku_sh,
               lsem0, lsem1, gsem0, gsem1, ssem0, ssem1):
    ksep_b = (ksep0, ksep1)
    ic_b = (ic0, ic1)
    rows_b = (rows0, rows1)
    cols_b = (cols0, cols1)
    uv_b = (uv0, uv1)
    lsem = (lsem0, lsem1)
    gsem = (gsem0, gsem1)
    ssem = (ssem0, ssem1)
    c = lax.axis_index("c")
    s = lax.axis_index("s")
    wid = s * NC + c

    pltpu.sync_copy(wx_hbm.at[pl.ds(wid * E_PER_W, E_PER_W)], wx_v)

    def scale_body(g, acc):
        x = wx_v[pl.ds(g * L, L)]
        rho = 1.0 / (1.0 + jnp.exp(-x))
        scale_v[pl.ds(g * L, L)] = EMIN + rho * rho * rho * (EMAX - EMIN)
        return acc + rho

    acc = lax.fori_loop(0, E_PER_W // L, scale_body,
                        jnp.zeros((L,), jnp.float32))
    rho_v[...] = acc
    pltpu.sync_copy(rho_v, rho_out.at[wid])

    def zero_body(i, _):
        zbuf_v[pl.ds(i * L, L)] = jnp.zeros((L,), jnp.float32)
        return 0

    lax.fori_loop(0, DOF_SLICE // L, zero_body, 0)
    pltpu.sync_copy(zbuf_v, ku_sh.at[pl.ds(s * DOF_SLICE, DOF_SLICE)])
    plsc.subcore_barrier()

    def lin_start(b, ch):
        rb = wid * ROWS_PER_W + ch * CH_ROWS
        pltpu.async_copy(ksep_hbm.at[pl.ds(rb, CH_ROWS)], ksep_b[b], lsem[b])
        pltpu.async_copy(ik_hbm.at[:, pl.ds(rb * 128, CH_ROWS * 128)],
                         ic_b[b], lsem[b])

    def lin_wait(b):
        pltpu.make_async_copy(
            ksep_hbm.at[pl.ds(0, CH_ROWS)], ksep_b[b], lsem[b]).wait()
        pltpu.make_async_copy(
            ik_hbm.at[:, pl.ds(0, CH_ROWS * 128)], ic_b[b], lsem[b]).wait()

    def g_fire(b):
        pltpu.async_copy(u_hbm.at[cols_b[b]], uv_b[b], gsem[b])

    def deint_idx(b):
        def body(k, _):
            rows_b[b][pl.ds(k * L, L)] = ic_b[b][0, pl.ds(k * L, L)]
            cols_b[b][pl.ds(k * L, L)] = ic_b[b][1, pl.ds(k * L, L)]
            return 0
        lax.fori_loop(0, CH_ROWS * 8, body, 0)

    def g_drain(b):
        pltpu.make_async_copy(
            ksep_hbm.at[pl.ds(0, CH_ROWS)], ksep_b[b], gsem[b]).wait()

    def s_fire(b):
        pltpu.async_copy(uv_b[b], ku_sh.at[rows_b[b]], ssem[b], add=True)

    def s_drain(b):
        pltpu.make_async_copy(
            ksep_hbm.at[pl.ds(0, CH_ROWS)], ksep_b[b], ssem[b]).wait()

    def compute(b, ch):
        def blk_body(blk, _):
            scale_vec = scale_v[pl.ds(2 * ch * CH_ROWS + blk * 16, 16)]
            for r8 in range(8):
                rr = blk * 8 + r8
                sc0 = jnp.take_along_axis(
                    scale_vec, jnp.full((L,), 2 * r8, jnp.int32),
                    axis=0, mode="promise_in_bounds")
                sc1 = jnp.take_along_axis(
                    scale_vec, jnp.full((L,), 2 * r8 + 1, jnp.int32),
                    axis=0, mode="promise_in_bounds")
                for j in range(8):
                    scv = sc0 if j < 4 else sc1
                    sl = pl.ds(rr * 128 + j * L, L)
                    uv_b[b][sl] = (ksep_b[b][rr, pl.ds(j * L, L)] * scv
                                   * uv_b[b][sl])
            return 0

        lax.fori_loop(0, CH_ROWS // 8, blk_body, 0)

    lin_start(0, 0)

    def pair_body(i, _):
        ch0 = 2 * i
        ch1 = 2 * i + 1
        lin_wait(0)
        deint_idx(0)
        g_fire(0)

        @pl.when(i > 0)
        def _():
            s_drain(1)

        lin_start(1, ch1)
        g_drain(0)
        compute(0, ch0)
        s_fire(0)
        lin_wait(1)
        deint_idx(1)
        g_fire(1)
        s_drain(0)

        @pl.when(ch0 + 2 < N_CH)
        def _():
            lin_start(0, ch0 + 2)

        g_drain(1)
        compute(1, ch1)
        s_fire(1)
        return 0

    lax.fori_loop(0, N_CH // 2, pair_body, 0)
    s_drain(1)
    plsc.subcore_barrier()

    base = s * DOF_SLICE
    pltpu.sync_copy(ku_sh.at[pl.ds(base, DOF_SLICE)], zbuf_v)
    pltpu.sync_copy(zbuf_v, ku_out.at[pl.ds(c * NDOF_PAD + base, DOF_SLICE)])


@functools.partial(
    pl.kernel,
    out_type=(jax.ShapeDtypeStruct((NC * NDOF_PAD,), jnp.float32),
              jax.ShapeDtypeStruct((NW, L), jnp.float32)),
    mesh=plsc.VectorSubcoreMesh(core_axis_name="c", subcore_axis_name="s",
                                num_cores=NC, num_subcores=NS),
    scratch_types=[
        pltpu.VMEM((E_PER_W,), jnp.float32),
        pltpu.VMEM((E_PER_W,), jnp.float32),
        pltpu.VMEM((CH_ROWS, 128), jnp.float32),
        pltpu.VMEM((2, CH_ROWS * 128), jnp.int32),
        pltpu.VMEM((CH_ROWS * 128,), jnp.int32),
        pltpu.VMEM((CH_ROWS * 128,), jnp.int32),
        pltpu.VMEM((CH_ROWS * 128,), jnp.float32),
        pltpu.VMEM((CH_ROWS, 128), jnp.float32),
        pltpu.VMEM((2, CH_ROWS * 128), jnp.int32),
        pltpu.VMEM((CH_ROWS * 128,), jnp.int32),
        pltpu.VMEM((CH_ROWS * 128,), jnp.int32),
        pltpu.VMEM((CH_ROWS * 128,), jnp.float32),
        pltpu.VMEM((DOF_SLICE,), jnp.float32),
        pltpu.VMEM((L,), jnp.float32),
        pltpu.VMEM_SHARED((NDOF_PAD,), jnp.float32),
        pltpu.SemaphoreType.DMA,
        pltpu.SemaphoreType.DMA,
        pltpu.SemaphoreType.DMA,
        pltpu.SemaphoreType.DMA,
        pltpu.SemaphoreType.DMA,
        pltpu.SemaphoreType.DMA,
    ],
)
def _sc_call(wx, ksep, ik, u, ku_out, rho_out, *scratch):
    _sc_kernel(wx, ksep, ik, u, ku_out, rho_out, *scratch)


def _tc_reduce(ku0_ref, ku1_ref, f_ref, rho_ref, out_ref):
    d = ku0_ref[...] + ku1_ref[...] - f_ref[...]
    ss = jnp.sum(d * d)
    rho_mean = jnp.sum(rho_ref[...]) / NME
    loss = jnp.maximum(rho_mean - VOLFRAC, 0.0) + jnp.sqrt(ss)
    out_ref[...] = jnp.broadcast_to(loss, (1, 1))


def kernel(W_x, K_sep, indeces_K, u, f):
    ksep2d = K_sep.reshape(ROWS_TOTAL, 128)
    ku_parts, rho_parts = _sc_call(W_x, ksep2d, indeces_K, u)

    f_pad = jnp.concatenate([f, jnp.zeros((NDOF_PAD - NDOF,), jnp.float32)])
    loss2d = pl.pallas_call(
        _tc_reduce,
        out_shape=jax.ShapeDtypeStruct((1, 1), jnp.float32),
    )(ku_parts[:NDOF_PAD].reshape(NDOF_PAD // 128, 128),
      ku_parts[NDOF_PAD:].reshape(NDOF_PAD // 128, 128),
      f_pad.reshape(NDOF_PAD // 128, 128),
      rho_parts.reshape(NW * L // 128, 128))
    return loss2d[0, 0]


# speedup vs baseline: 1.1702x; 1.1702x over previous
import functools

import jax
import jax.numpy as jnp
from jax import lax
from jax.experimental import pallas as pl
from jax.experimental.pallas import tpu as pltpu
from jax.experimental.pallas import tpu_sc as plsc

NME = 65536
NNZ_PER = 64
NNZ = NME * NNZ_PER
NDOF = 132098
NDOF_PAD = 132608
EMIN = 1e-09
EMAX = 1.0
PENAL = 3.0
VOLFRAC = 0.4

NC = 2
NS = 16
NW = NC * NS
L = 16

ROWS_TOTAL = NNZ // 128
ROWS_PER_W = ROWS_TOTAL // NW
CH_ROWS = 64
N_CH = ROWS_PER_W // CH_ROWS
E_PER_W = NME // NW
DOF_SLICE = NDOF_PAD // NS


def _sc_kernel(wx_hbm, ksep_hbm, ik_hbm, colsf_hbm, u_hbm,
               ku_out, rho_out,
               wx_v, scale_v,
               ksep0, rows0, cols0, uv0,
               ksep1, rows1, cols1, uv1,
               zbuf_v, rho_v, ku_sh,
               lsem0, lsem1, gsem0, gsem1, ssem0, ssem1):
    ksep_b = (ksep0, ksep1)
    rows_b = (rows0, rows1)
    cols_b = (cols0, cols1)
    uv_b = (uv0, uv1)
    lsem = (lsem0, lsem1)
    gsem = (gsem0, gsem1)
    ssem = (ssem0, ssem1)
    c = lax.axis_index("c")
    s = lax.axis_index("s")
    wid = s * NC + c

    pltpu.sync_copy(wx_hbm.at[pl.ds(wid * E_PER_W, E_PER_W)], wx_v)

    def scale_body(g, acc):
        x = wx_v[pl.ds(g * L, L)]
        rho = 1.0 / (1.0 + jnp.exp(-x))
        scale_v[pl.ds(g * L, L)] = EMIN + rho * rho * rho * (EMAX - EMIN)
        return acc + rho

    acc = lax.fori_loop(0, E_PER_W // L, scale_body,
                        jnp.zeros((L,), jnp.float32))
    rho_v[...] = acc
    pltpu.sync_copy(rho_v, rho_out.at[wid])

    def zero_body(i, _):
        zbuf_v[pl.ds(i * L, L)] = jnp.zeros((L,), jnp.float32)
        return 0

    lax.fori_loop(0, DOF_SLICE // L, zero_body, 0)
    pltpu.sync_copy(zbuf_v, ku_sh.at[pl.ds(s * DOF_SLICE, DOF_SLICE)])
    plsc.subcore_barrier()

    def lin_start(b, ch):
        rb = wid * ROWS_PER_W + ch * CH_ROWS
        pltpu.async_copy(ksep_hbm.at[pl.ds(rb, CH_ROWS)], ksep_b[b], lsem[b])
        pltpu.async_copy(ik_hbm.at[0, pl.ds(rb * 128, CH_ROWS * 128)],
                         rows_b[b], lsem[b])
        pltpu.async_copy(colsf_hbm.at[pl.ds(rb * 128, CH_ROWS * 128)],
                         cols_b[b], lsem[b])

    def lin_wait(b):
        pltpu.make_async_copy(
            ksep_hbm.at[pl.ds(0, CH_ROWS)], ksep_b[b], lsem[b]).wait()
        pltpu.make_async_copy(
            ik_hbm.at[0, pl.ds(0, CH_ROWS * 128)], rows_b[b], lsem[b]).wait()
        pltpu.make_async_copy(
            colsf_hbm.at[pl.ds(0, CH_ROWS * 128)], cols_b[b], lsem[b]).wait()

    def g_fire(b):
        pltpu.async_copy(u_hbm.at[cols_b[b]], uv_b[b], gsem[b])

    def g_drain(b):
        pltpu.make_async_copy(
            ksep_hbm.at[pl.ds(0, CH_ROWS)], ksep_b[b], gsem[b]).wait()

    def s_fire(b):
        pltpu.async_copy(uv_b[b], ku_sh.at[rows_b[b]], ssem[b], add=True)

    def s_drain(b):
        pltpu.make_async_copy(
            ksep_hbm.at[pl.ds(0, CH_ROWS)], ksep_b[b], ssem[b]).wait()

    def compute(b, ch):
        def blk_body(blk, _):
            scale_vec = scale_v[pl.ds(2 * ch * CH_ROWS + blk * 16, 16)]
            for r8 in range(8):
                rr = blk * 8 + r8
                sc0 = jnp.take_along_axis(
                    scale_vec, jnp.full((L,), 2 * r8, jnp.int32),
                    axis=0, mode="promise_in_bounds")
                sc1 = jnp.take_along_axis(
                    scale_vec, jnp.full((L,), 2 * r8 + 1, jnp.int32),
                    axis=0, mode="promise_in_bounds")
                for j in range(8):
                    scv = sc0 if j < 4 else sc1
                    sl = pl.ds(rr * 128 + j * L, L)
                    uv_b[b][sl] = (ksep_b[b][rr, pl.ds(j * L, L)] * scv
                                   * uv_b[b][sl])
            return 0

        lax.fori_loop(0, CH_ROWS // 8, blk_body, 0)

    lin_start(0, 0)

    def pair_body(i, _):
        ch0 = 2 * i
        ch1 = 2 * i + 1
        lin_wait(0)
        g_fire(0)

        @pl.when(i > 0)
        def _():
            s_drain(1)

        lin_start(1, ch1)
        g_drain(0)
        compute(0, ch0)
        s_fire(0)
        lin_wait(1)
        g_fire(1)
        s_drain(0)

        @pl.when(ch0 + 2 < N_CH)
        def _():
            lin_start(0, ch0 + 2)

        g_drain(1)
        compute(1, ch1)
        s_fire(1)
        return 0

    lax.fori_loop(0, N_CH // 2, pair_body, 0)
    s_drain(1)
    plsc.subcore_barrier()

    base = s * DOF_SLICE
    pltpu.sync_copy(ku_sh.at[pl.ds(base, DOF_SLICE)], zbuf_v)
    pltpu.sync_copy(zbuf_v, ku_out.at[pl.ds(c * NDOF_PAD + base, DOF_SLICE)])


@functools.partial(
    pl.kernel,
    out_type=(jax.ShapeDtypeStruct((NC * NDOF_PAD,), jnp.float32),
              jax.ShapeDtypeStruct((NW, L), jnp.float32)),
    mesh=plsc.VectorSubcoreMesh(core_axis_name="c", subcore_axis_name="s",
                                num_cores=NC, num_subcores=NS),
    scratch_types=[
        pltpu.VMEM((E_PER_W,), jnp.float32),
        pltpu.VMEM((E_PER_W,), jnp.float32),
        pltpu.VMEM((CH_ROWS, 128), jnp.float32),
        pltpu.VMEM((CH_ROWS * 128,), jnp.int32),
        pltpu.VMEM((CH_ROWS * 128,), jnp.int32),
        pltpu.VMEM((CH_ROWS * 128,), jnp.float32),
        pltpu.VMEM((CH_ROWS, 128), jnp.float32),
        pltpu.VMEM((CH_ROWS * 128,), jnp.int32),
        pltpu.VMEM((CH_ROWS * 128,), jnp.int32),
        pltpu.VMEM((CH_ROWS * 128,), jnp.float32),
        pltpu.VMEM((DOF_SLICE,), jnp.float32),
        pltpu.VMEM((L,), jnp.float32),
        pltpu.VMEM_SHARED((NDOF_PAD,), jnp.float32),
        pltpu.SemaphoreType.DMA,
        pltpu.SemaphoreType.DMA,
        pltpu.SemaphoreType.DMA,
        pltpu.SemaphoreType.DMA,
        pltpu.SemaphoreType.DMA,
        pltpu.SemaphoreType.DMA,
    ],
)
def _sc_call(wx, ksep, ik, colsf, u, ku_out, rho_out, *scratch):
    _sc_kernel(wx, ksep, ik, colsf, u, ku_out, rho_out, *scratch)


def _tc_reduce(ku0_ref, ku1_ref, f_ref, rho_ref, out_ref):
    d = ku0_ref[...] + ku1_ref[...] - f_ref[...]
    ss = jnp.sum(d * d)
    rho_mean = jnp.sum(rho_ref[...]) / NME
    loss = jnp.maximum(rho_mean - VOLFRAC, 0.0) + jnp.sqrt(ss)
    out_ref[...] = jnp.broadcast_to(loss, (1, 1))


def kernel(W_x, K_sep, indeces_K, u, f):
    ksep2d = K_sep.reshape(ROWS_TOTAL, 128)
    cols_flat = indeces_K[1]
    ku_parts, rho_parts = _sc_call(W_x, ksep2d, indeces_K, cols_flat, u)

    f_pad = jnp.concatenate([f, jnp.zeros((NDOF_PAD - NDOF,), jnp.float32)])
    loss2d = pl.pallas_call(
        _tc_reduce,
        out_shape=jax.ShapeDtypeStruct((1, 1), jnp.float32),
    )(ku_parts[:NDOF_PAD].reshape(NDOF_PAD // 128, 128),
      ku_parts[NDOF_PAD:].reshape(NDOF_PAD // 128, 128),
      f_pad.reshape(NDOF_PAD // 128, 128),
      rho_parts.reshape(NW * L // 128, 128))
    return loss2d[0, 0]


# speedup vs baseline: 1.8447x; 1.5764x over previous
import functools

import jax
import jax.numpy as jnp
from jax import lax
from jax.experimental import pallas as pl
from jax.experimental.pallas import tpu as pltpu
from jax.experimental.pallas import tpu_sc as plsc

NME = 65536
NNZ_PER = 64
NNZ = NME * NNZ_PER
NDOF = 132098
NDOF_PAD = 132608
EMIN = 1e-09
EMAX = 1.0
PENAL = 3.0
VOLFRAC = 0.4

NC = 2
NS = 16
NW = NC * NS
L = 16

ROWS_TOTAL = NNZ // 128
ROWS_PER_W = ROWS_TOTAL // NW
CH_ROWS = 64
N_CH = ROWS_PER_W // CH_ROWS
E_PER_W = NME // NW
DOF_SLICE = NDOF_PAD // NS


def _sc_kernel(wx_hbm, ksep_hbm, ik_hbm, colsf_hbm, u_hbm,
               ku_out, rho_out,
               wx_v, scale_v,
               ksep0, rows0, cols0, uv0,
               ksep1, rows1, cols1, uv1,
               zbuf_v, rho_v, ku_sh, u_sh,
               lsem0, lsem1, gsem0, gsem1, ssem0, ssem1):
    ksep_b = (ksep0, ksep1)
    rows_b = (rows0, rows1)
    cols_b = (cols0, cols1)
    uv_b = (uv0, uv1)
    lsem = (lsem0, lsem1)
    gsem = (gsem0, gsem1)
    ssem = (ssem0, ssem1)
    c = lax.axis_index("c")
    s = lax.axis_index("s")
    wid = s * NC + c

    pltpu.sync_copy(wx_hbm.at[pl.ds(wid * E_PER_W, E_PER_W)], wx_v)

    def scale_body(g, acc):
        x = wx_v[pl.ds(g * L, L)]
        rho = 1.0 / (1.0 + jnp.exp(-x))
        scale_v[pl.ds(g * L, L)] = EMIN + rho * rho * rho * (EMAX - EMIN)
        return acc + rho

    acc = lax.fori_loop(0, E_PER_W // L, scale_body,
                        jnp.zeros((L,), jnp.float32))
    rho_v[...] = acc
    pltpu.sync_copy(rho_v, rho_out.at[wid])

    def zero_body(i, _):
        zbuf_v[pl.ds(i * L, L)] = jnp.zeros((L,), jnp.float32)
        return 0

    lax.fori_loop(0, DOF_SLICE // L, zero_body, 0)
    pltpu.sync_copy(zbuf_v, ku_sh.at[pl.ds(s * DOF_SLICE, DOF_SLICE)])
    u_lo = s * DOF_SLICE
    pltpu.sync_copy(u_hbm.at[pl.ds(u_lo, DOF_SLICE)], zbuf_v)
    pltpu.sync_copy(zbuf_v, u_sh.at[pl.ds(u_lo, DOF_SLICE)])
    plsc.subcore_barrier()

    def lin_start(b, ch):
        rb = wid * ROWS_PER_W + ch * CH_ROWS
        pltpu.async_copy(ksep_hbm.at[pl.ds(rb, CH_ROWS)], ksep_b[b], lsem[b])
        pltpu.async_copy(ik_hbm.at[0, pl.ds(rb * 128, CH_ROWS * 128)],
                         rows_b[b], lsem[b])
        pltpu.async_copy(colsf_hbm.at[pl.ds(rb * 128, CH_ROWS * 128)],
                         cols_b[b], lsem[b])

    def lin_wait(b):
        pltpu.make_async_copy(
            ksep_hbm.at[pl.ds(0, CH_ROWS)], ksep_b[b], lsem[b]).wait()
        pltpu.make_async_copy(
            ik_hbm.at[0, pl.ds(0, CH_ROWS * 128)], rows_b[b], lsem[b]).wait()
        pltpu.make_async_copy(
            colsf_hbm.at[pl.ds(0, CH_ROWS * 128)], cols_b[b], lsem[b]).wait()

    def g_fire(b):
        pltpu.async_copy(u_sh.at[cols_b[b]], uv_b[b], gsem[b])

    def g_drain(b):
        pltpu.make_async_copy(
            ksep_hbm.at[pl.ds(0, CH_ROWS)], ksep_b[b], gsem[b]).wait()

    def s_fire(b):
        pltpu.async_copy(uv_b[b], ku_sh.at[rows_b[b]], ssem[b], add=True)

    def s_drain(b):
        pltpu.make_async_copy(
            ksep_hbm.at[pl.ds(0, CH_ROWS)], ksep_b[b], ssem[b]).wait()

    def compute(b, ch):
        def blk_body(blk, _):
            scale_vec = scale_v[pl.ds(2 * ch * CH_ROWS + blk * 16, 16)]
            for r8 in range(8):
                rr = blk * 8 + r8
                sc0 = jnp.take_along_axis(
                    scale_vec, jnp.full((L,), 2 * r8, jnp.int32),
                    axis=0, mode="promise_in_bounds")
                sc1 = jnp.take_along_axis(
                    scale_vec, jnp.full((L,), 2 * r8 + 1, jnp.int32),
                    axis=0, mode="promise_in_bounds")
                for j in range(8):
                    scv = sc0 if j < 4 else sc1
                    sl = pl.ds(rr * 128 + j * L, L)
                    uv_b[b][sl] = (ksep_b[b][rr, pl.ds(j * L, L)] * scv
                                   * uv_b[b][sl])
            return 0

        lax.fori_loop(0, CH_ROWS // 8, blk_body, 0)

    lin_start(0, 0)

    def pair_body(i, _):
        ch0 = 2 * i
        ch1 = 2 * i + 1
        lin_wait(0)
        g_fire(0)

        @pl.when(i > 0)
        def _():
            s_drain(1)

        lin_start(1, ch1)
        g_drain(0)
        compute(0, ch0)
        s_fire(0)
        lin_wait(1)
        g_fire(1)
        s_drain(0)

        @pl.when(ch0 + 2 < N_CH)
        def _():
            lin_start(0, ch0 + 2)

        g_drain(1)
        compute(1, ch1)
        s_fire(1)
        return 0

    lax.fori_loop(0, N_CH // 2, pair_body, 0)
    s_drain(1)
    plsc.subcore_barrier()

    base = s * DOF_SLICE
    pltpu.sync_copy(ku_sh.at[pl.ds(base, DOF_SLICE)], zbuf_v)
    pltpu.sync_copy(zbuf_v, ku_out.at[pl.ds(c * NDOF_PAD + base, DOF_SLICE)])


@functools.partial(
    pl.kernel,
    out_type=(jax.ShapeDtypeStruct((NC * NDOF_PAD,), jnp.float32),
              jax.ShapeDtypeStruct((NW, L), jnp.float32)),
    mesh=plsc.VectorSubcoreMesh(core_axis_name="c", subcore_axis_name="s",
                                num_cores=NC, num_subcores=NS),
    scratch_types=[
        pltpu.VMEM((E_PER_W,), jnp.float32),
        pltpu.VMEM((E_PER_W,), jnp.float32),
        pltpu.VMEM((CH_ROWS, 128), jnp.float32),
        pltpu.VMEM((CH_ROWS * 128,), jnp.int32),
        pltpu.VMEM((CH_ROWS * 128,), jnp.int32),
        pltpu.VMEM((CH_ROWS * 128,), jnp.float32),
        pltpu.VMEM((CH_ROWS, 128), jnp.float32),
        pltpu.VMEM((CH_ROWS * 128,), jnp.int32),
        pltpu.VMEM((CH_ROWS * 128,), jnp.int32),
        pltpu.VMEM((CH_ROWS * 128,), jnp.float32),
        pltpu.VMEM((DOF_SLICE,), jnp.float32),
        pltpu.VMEM((L,), jnp.float32),
        pltpu.VMEM_SHARED((NDOF_PAD,), jnp.float32),
        pltpu.VMEM_SHARED((NDOF_PAD,), jnp.float32),
        pltpu.SemaphoreType.DMA,
        pltpu.SemaphoreType.DMA,
        pltpu.SemaphoreType.DMA,
        pltpu.SemaphoreType.DMA,
        pltpu.SemaphoreType.DMA,
        pltpu.SemaphoreType.DMA,
    ],
)
def _sc_call(wx, ksep, ik, colsf, u, ku_out, rho_out, *scratch):
    _sc_kernel(wx, ksep, ik, colsf, u, ku_out, rho_out, *scratch)


def _tc_reduce(ku0_ref, ku1_ref, f_ref, rho_ref, out_ref):
    d = ku0_ref[...] + ku1_ref[...] - f_ref[...]
    ss = jnp.sum(d * d)
    rho_mean = jnp.sum(rho_ref[...]) / NME
    loss = jnp.maximum(rho_mean - VOLFRAC, 0.0) + jnp.sqrt(ss)
    out_ref[...] = jnp.broadcast_to(loss, (1, 1))


def kernel(W_x, K_sep, indeces_K, u, f):
    ksep2d = K_sep.reshape(ROWS_TOTAL, 128)
    cols_flat = indeces_K[1]
    u_pad = jnp.concatenate([u, jnp.zeros((NDOF_PAD - NDOF,), jnp.float32)])
    ku_parts, rho_parts = _sc_call(W_x, ksep2d, indeces_K, cols_flat, u_pad)

    f_pad = jnp.concatenate([f, jnp.zeros((NDOF_PAD - NDOF,), jnp.float32)])
    loss2d = pl.pallas_call(
        _tc_reduce,
        out_shape=jax.ShapeDtypeStruct((1, 1), jnp.float32),
    )(ku_parts[:NDOF_PAD].reshape(NDOF_PAD // 128, 128),
      ku_parts[NDOF_PAD:].reshape(NDOF_PAD // 128, 128),
      f_pad.reshape(NDOF_PAD // 128, 128),
      rho_parts.reshape(NW * L // 128, 128))
    return loss2d[0, 0]
